# baseline (speedup 1.0000x reference)
import jax
import jax.numpy as jnp
import numpy as np
from jax import lax
from jax.experimental import pallas as pl
from jax.experimental.pallas import tpu as pltpu
from jax.experimental.pallas import tpu_sc as plsc

N_ROWS = 16384
N_COLS = 128

with jax.default_device(jax.local_devices(backend="cpu")[0]):
    _PERM = np.asarray(
        jax.random.permutation(jax.random.key(1), N_ROWS), dtype=np.int32)

_info = plsc.get_sparse_core_info()
_NC, _NS, _L = _info.num_cores, _info.num_subcores, _info.num_lanes
_NW = _NC * _NS
_ROWS_PER_W = N_ROWS // _NW
_CHUNK = 128
_NCHUNK = _ROWS_PER_W // _CHUNK
_NBUF = 3


def _body(x_hbm, perm_hbm, out_hbm,
          idx_v, gat_v, org_v,
          isem0, isem1, isem2, osem0, osem1, osem2):
    wid = lax.axis_index("s") * _NC + lax.axis_index("c")
    base = wid * _ROWS_PER_W
    lane = lax.iota(jnp.int32, _L)
    even = (lane % 2) == 0

    gat = [gat_v.at[s] for s in range(_NBUF)]
    org = [org_v.at[s] for s in range(_NBUF)]
    isem = [isem0, isem1, isem2]
    osem = [osem0, osem1, osem2]

    pltpu.sync_copy(perm_hbm.at[pl.ds(wid * _NCHUNK, _NCHUNK)], idx_v)

    def start_in(c, s):
        row0 = base + c * _CHUNK
        g = pltpu.async_copy(x_hbm.at[idx_v.at[c]], gat[s], isem[s])
        o = pltpu.async_copy(x_hbm.at[pl.ds(row0, _CHUNK)], org[s], isem[s])
        return g, o

    def merge(s, lo, hi):
        g_buf, o_buf = gat[s], org[s]

        @plsc.parallel_loop(lo, hi, step=1, unroll=2)
        def row_body(r):
            for cc in range(N_COLS // _L):
                sl = pl.ds(cc * _L, _L)
                gv = g_buf[r, sl]
                ov = o_buf[r, sl]
                o_buf[r, sl] = jnp.where(even, gv, ov)

    in_flight = {}
    out_flight = {}
    for c in range(min(_NBUF, _NCHUNK)):
        in_flight[c] = start_in(c, c % _NBUF)
    for c in range(_NCHUNK):
        s = c % _NBUF
        gd, cd = in_flight.pop(c)
        gd.wait()
        cd.wait()
        merge(s, 0, _CHUNK)
        row0 = base + c * _CHUNK
        out_flight[c] = pltpu.async_copy(
            org[s], out_hbm.at[pl.ds(row0, _CHUNK)], osem[s])
        nxt = c + _NBUF
        if nxt < _NCHUNK:
            out_flight.pop(c).wait()
            in_flight[nxt] = start_in(nxt, s)
    for c in sorted(out_flight):
        out_flight[c].wait()


@jax.jit
def kernel(x):
    mesh = plsc.VectorSubcoreMesh(core_axis_name="c", subcore_axis_name="s")
    perm = jnp.asarray(_PERM).reshape(N_ROWS // N_COLS, N_COLS)
    run = pl.kernel(
        _body,
        out_type=jax.ShapeDtypeStruct((N_ROWS, N_COLS), jnp.float32),
        mesh=mesh,
        compiler_params=pltpu.CompilerParams(use_tc_tiling_on_sc=True),
        scratch_types=[
            pltpu.VMEM((_NCHUNK, N_COLS), jnp.int32),
            pltpu.VMEM((_NBUF, _CHUNK, N_COLS), jnp.float32),
            pltpu.VMEM((_NBUF, _CHUNK, N_COLS), jnp.float32),
            pltpu.SemaphoreType.DMA,
            pltpu.SemaphoreType.DMA,
            pltpu.SemaphoreType.DMA,
            pltpu.SemaphoreType.DMA,
            pltpu.SemaphoreType.DMA,
            pltpu.SemaphoreType.DMA,
        ],
    )
    return run(x, perm)


# speedup vs baseline: 1.0316x; 1.0316x over previous
import jax
import jax.numpy as jnp
import numpy as np
from jax import lax
from jax.experimental import pallas as pl
from jax.experimental.pallas import tpu as pltpu
from jax.experimental.pallas import tpu_sc as plsc

N_ROWS = 16384
N_COLS = 128

with jax.default_device(jax.local_devices(backend="cpu")[0]):
    _PERM = np.asarray(
        jax.random.permutation(jax.random.key(1), N_ROWS), dtype=np.int32)

_info = plsc.get_sparse_core_info()
_NC, _NS, _L = _info.num_cores, _info.num_subcores, _info.num_lanes
_NW = _NC * _NS
_ROWS_PER_W = N_ROWS // _NW
_CHUNK = 128
_NCHUNK = _ROWS_PER_W // _CHUNK
_NBUF = 3


def _body(x_hbm, perm_hbm, out_hbm,
          idx_v, gat_v, org_v,
          gsem0, gsem1, gsem2, csem0, csem1, csem2, osem0, osem1, osem2):
    wid = lax.axis_index("s") * _NC + lax.axis_index("c")
    base = wid * _ROWS_PER_W
    lane = lax.iota(jnp.int32, _L)
    even = (lane % 2) == 0

    gat = [gat_v.at[s] for s in range(_NBUF)]
    org = [org_v.at[s] for s in range(_NBUF)]
    gsem = [gsem0, gsem1, gsem2]
    csem = [csem0, csem1, csem2]
    osem = [osem0, osem1, osem2]

    pltpu.sync_copy(perm_hbm.at[pl.ds(wid * _NCHUNK, _NCHUNK)], idx_v)

    def start_in(c, s):
        row0 = base + c * _CHUNK
        g = pltpu.async_copy(x_hbm.at[idx_v.at[c]], gat[s], gsem[s])
        o = pltpu.async_copy(x_hbm.at[pl.ds(row0, _CHUNK)], org[s], csem[s])
        return g, o

    def merge(s, lo, hi):
        g_buf, o_buf = gat[s], org[s]

        @plsc.parallel_loop(lo, hi, step=1, unroll=2)
        def row_body(r):
            for cc in range(N_COLS // _L):
                sl = pl.ds(cc * _L, _L)
                gv = g_buf[r, sl]
                ov = o_buf[r, sl]
                o_buf[r, sl] = jnp.where(even, gv, ov)

    in_flight = {}
    out_flight = {}
    for c in range(min(_NBUF, _NCHUNK)):
        in_flight[c] = start_in(c, c % _NBUF)
    for c in range(_NCHUNK):
        s = c % _NBUF
        gd, cd = in_flight.pop(c)
        gd.wait()
        cd.wait()
        merge(s, 0, _CHUNK)
        row0 = base + c * _CHUNK
        out_flight[c] = pltpu.async_copy(
            org[s], out_hbm.at[pl.ds(row0, _CHUNK)], osem[s])
        nxt = c + _NBUF
        if nxt < _NCHUNK:
            out_flight.pop(c).wait()
            in_flight[nxt] = start_in(nxt, s)
    for c in sorted(out_flight):
        out_flight[c].wait()


@jax.jit
def kernel(x):
    mesh = plsc.VectorSubcoreMesh(core_axis_name="c", subcore_axis_name="s")
    perm = jnp.asarray(_PERM).reshape(N_ROWS // N_COLS, N_COLS)
    run = pl.kernel(
        _body,
        out_type=jax.ShapeDtypeStruct((N_ROWS, N_COLS), jnp.float32),
        mesh=mesh,
        compiler_params=pltpu.CompilerParams(use_tc_tiling_on_sc=True),
        scratch_types=[
            pltpu.VMEM((_NCHUNK, N_COLS), jnp.int32),
            pltpu.VMEM((_NBUF, _CHUNK, N_COLS), jnp.float32),
            pltpu.VMEM((_NBUF, _CHUNK, N_COLS), jnp.float32),
            pltpu.SemaphoreType.DMA,
            pltpu.SemaphoreType.DMA,
            pltpu.SemaphoreType.DMA,
            pltpu.SemaphoreType.DMA,
            pltpu.SemaphoreType.DMA,
            pltpu.SemaphoreType.DMA,
            pltpu.SemaphoreType.DMA,
            pltpu.SemaphoreType.DMA,
            pltpu.SemaphoreType.DMA,
        ],
    )
    return run(x, perm)
